# Initial kernel scaffold; baseline (speedup 1.0000x reference)
import jax
import jax.numpy as jnp
from jax.experimental import pallas as pl


def kernel(x, state, Wk, Wq, Wv, gamma, beta, W1, b1, W2, b2, Ws, bs):
    raise NotImplementedError("write your pallas kernel here")


# speedup vs baseline: 92.7680x; 92.7680x over previous
import functools

import jax
import jax.numpy as jnp
import numpy as np
from jax.experimental import pallas as pl
from jax.experimental.pallas import tpu as pltpu

EPS_LN = 1e-5
EPS_SUMNORM = 1e-5
B, T, F, H = 8, 1024, 128, 64
C = 256


def _fwp_kernel(x_ref, state_f_ref, state_m_ref, wk_ref, wq_ref, wv_ref,
                g_ref, beta_ref, w1_ref, b1_ref, w2_ref, b2_ref, ws_ref,
                bs_ref, ei_ref, l16_ref, l32_ref,
                y_ref, s_ref, cf_ref, cm_ref):
    tc = pl.program_id(1)

    @pl.when(tc == 0)
    def _init():
        cf_ref[...] = state_f_ref[0]
        cm_ref[...] = state_m_ref[0]

    x = x_ref[0]
    mu = jnp.mean(x, axis=1, keepdims=True)
    xc = x - mu
    var = jnp.mean(xc * xc, axis=1, keepdims=True)
    xn = xc * jax.lax.rsqrt(var + EPS_LN) * g_ref[...] + beta_ref[...]

    dg_t = lambda a, w: jax.lax.dot_general(
        a, w, (((1,), (1,)), ((), ())), preferred_element_type=jnp.float32)
    K = jnp.maximum(dg_t(xn, wk_ref[...]), 0.0)
    Q = jnp.maximum(dg_t(xn, wq_ref[...]), 0.0)
    V = dg_t(xn, wv_ref[...])
    K = K / (EPS_SUMNORM + jnp.sum(K, axis=1, keepdims=True))
    Q = Q / (EPS_SUMNORM + jnp.sum(Q, axis=1, keepdims=True))

    v_rep = jax.lax.dot_general(
        V.astype(jnp.bfloat16), ei_ref[...], (((1,), (0,)), ((), ())),
        preferred_element_type=jnp.float32)
    k2 = jnp.concatenate([K, K], axis=1)
    k_tile = jnp.tile(k2, (1, H // 2))
    kv16 = (v_rep * k_tile).astype(jnp.bfloat16)

    s_intra = jax.lax.dot_general(
        l16_ref[...], kv16, (((1,), (0,)), ((), ())),
        preferred_element_type=jnp.float32)
    s_blk = s_intra + cf_ref[...]
    s_ref[0] = s_blk

    a = dg_t(Q, K) * l32_ref[...]
    y_intra = jax.lax.dot_general(
        a, V, (((1,), (0,)), ((), ())), preferred_element_type=jnp.float32)
    y_base = dg_t(Q, cm_ref[...])
    y = y_base + y_intra

    h = jnp.maximum(dg_t(y, w1_ref[...]) + b1_ref[...], 0.0)
    h = jnp.maximum(dg_t(h, w2_ref[...]) + b2_ref[...], 0.0)
    y_ref[0] = h + dg_t(xn, ws_ref[...]) + bs_ref[...]

    cf_ref[...] = s_blk[C - 1:C, :]
    cm_ref[...] = cm_ref[...] + jax.lax.dot_general(
        V, K, (((0,), (0,)), ((), ())), preferred_element_type=jnp.float32)


@jax.jit
def kernel(x, state, Wk, Wq, Wv, gamma, beta, W1, b1, W2, b2, Ws, bs):
    col = np.arange(H * H)
    ei = np.zeros((H, H * H), np.float32)
    ei[col // H, col] = 1.0
    ei = jnp.asarray(ei, dtype=jnp.bfloat16)
    ltri = np.tril(np.ones((C, C), np.float32))
    l16 = jnp.asarray(ltri, dtype=jnp.bfloat16)
    l32 = jnp.asarray(ltri)

    state_f = state.reshape(B, 1, H * H)
    state_m = state.reshape(B, H, H)

    full = lambda shp: pl.BlockSpec(shp, lambda b, t: (0,) * len(shp))
    in_specs = [
        pl.BlockSpec((1, C, F), lambda b, t: (b, t, 0)),
        pl.BlockSpec((1, 1, H * H), lambda b, t: (b, 0, 0)),
        pl.BlockSpec((1, H, H), lambda b, t: (b, 0, 0)),
        full((H, F)), full((H, F)), full((H, F)),
        full((1, F)), full((1, F)),
        full((H, H)), full((1, H)),
        full((H, H)), full((1, H)),
        full((H, F)), full((1, H)),
        full((H, H * H)),
        full((C, C)), full((C, C)),
    ]
    out_specs = [
        pl.BlockSpec((1, C, H), lambda b, t: (b, t, 0)),
        pl.BlockSpec((1, C, H * H), lambda b, t: (b, t, 0)),
    ]
    y, s_flat = pl.pallas_call(
        _fwp_kernel,
        grid=(B, T // C),
        in_specs=in_specs,
        out_specs=out_specs,
        out_shape=[
            jax.ShapeDtypeStruct((B, T, H), jnp.float32),
            jax.ShapeDtypeStruct((B, T, H * H), jnp.float32),
        ],
        scratch_shapes=[
            pltpu.VMEM((1, H * H), jnp.float32),
            pltpu.VMEM((H, H), jnp.float32),
        ],
        compiler_params=pltpu.CompilerParams(
            dimension_semantics=("parallel", "arbitrary"),
            vmem_limit_bytes=60 * 1024 * 1024,
        ),
    )(x, state_f, state_m, Wk, Wq, Wv, gamma.reshape(1, F),
      beta.reshape(1, F), W1, b1.reshape(1, H), W2, b2.reshape(1, H),
      Ws, bs.reshape(1, H), ei, l16, l32)
    return y, s_flat.reshape(B, T, H, H)
